# baseline (speedup 1.0000x reference)
import functools

import jax
import jax.numpy as jnp
from jax import lax
from jax.experimental import pallas as pl
from jax.experimental.pallas import tpu as pltpu
from jax.experimental.pallas import tpu_sc as plsc

N = 20000
LANES = 128
ROWS = 160
NPAD = ROWS * LANES
K = 2000
KPAD = 2048
BLK = 256
NCH = 16
POST = 100
NEG = -3.0e38
NW = 32
EPW = NPAD // NW
R_OFF = KPAD + 8
OUT_ROWS = R_OFF + KPAD + 16

f32 = jnp.float32
i32 = jnp.int32
u32 = jnp.uint32

C_XG, C_YG, C_ZG, C_WG, C_LG, C_HG, C_RG, C_RW = 0, 1, 2, 3, 4, 5, 6, 7
C_X2, C_Y2, C_RA, C_ST, C_SR, C_IDX = 8, 9, 10, 11, 12, 13


def _sortkey(s, valid):
    u = lax.bitcast_convert_type(s, u32)
    key = jnp.where(u >= u32(0x80000000), ~u, u | u32(0x80000000))
    return jnp.where(valid, key, u32(0))


def _kth_key2(key_a, key_b, kwant):
    ta = u32(0)
    tb = u32(0)
    for b in range(31, -1, -1):
        ca = ta | u32(1 << b)
        cb = tb | u32(1 << b)
        cnta = jnp.sum((key_a >= ca).astype(i32))
        cntb = jnp.sum((key_b >= cb).astype(i32))
        ta = jnp.where(cnta >= kwant, ca, ta)
        tb = jnp.where(cntb >= kwant, cb, tb)
    return ta, tb


def _excl_cumsum(x, l_incl, l_strict_rows):
    incl = jnp.dot(x, l_incl, preferred_element_type=f32)
    rowtot = jnp.sum(x, axis=1, keepdims=True)
    prev_rows = jnp.dot(l_strict_rows, rowtot, preferred_element_type=f32)
    return incl - x + prev_rows


def _dest_of(key, t, l_incl, l_strict_rows, dump):
    gt = key > t
    eq = key == t
    c_gt = jnp.sum(gt.astype(f32))
    need = f32(K) - c_gt
    tie_rank = _excl_cumsum(eq.astype(f32), l_incl, l_strict_rows)
    sel = gt | (eq & (tie_rank < need))
    pos = _excl_cumsum(sel.astype(f32), l_incl, l_strict_rows).astype(i32)
    return jnp.where(sel, pos, i32(dump))


def _prep_body(resid_ref, logit_ref, res_ref, reg_ref, anc_ref,
               table_ref, destt_ref, destr_ref):
    resid = resid_ref[0, 0]
    logit = logit_ref[...]
    res = res_ref[...]
    xa, ya, za = anc_ref[0], anc_ref[1], anc_ref[2]
    wa, la, ha, ra = anc_ref[3], anc_ref[4], anc_ref[5], anc_ref[6]
    r0, r1, r2 = reg_ref[0], reg_ref[1], reg_ref[2]
    r3, r4, r5, r6 = reg_ref[3], reg_ref[4], reg_ref[5], reg_ref[6]

    diag = jnp.sqrt(la * la + wa * wa)
    xg = r0 / 10.0 * diag + xa
    yg = r1 / 10.0 * diag + ya
    zg = r2 / 10.0 * ha + za
    wg = jnp.exp(r3 / 5.0) * wa
    lg = jnp.exp(r4 / 5.0) * la
    hg = jnp.exp(r5 / 5.0) * ha
    rg = r6 / 10.0 + ra
    rw = jnp.arctan2(jnp.sin(rg), jnp.cos(rg))
    x2 = r0 / 10.0 * wa + xa
    y2 = r1 / 10.0 * la + ya

    score_t = jax.nn.sigmoid(jax.nn.sigmoid(logit)) + resid
    score_r = res + resid

    row_i = lax.broadcasted_iota(i32, (ROWS, LANES), 0)
    lane_i = lax.broadcasted_iota(i32, (ROWS, LANES), 1)
    flat = row_i * LANES + lane_i
    valid = flat < N
    flat_f = flat.astype(f32)

    for c, v in ((C_XG, xg), (C_YG, yg), (C_ZG, zg), (C_WG, wg), (C_LG, lg),
                 (C_HG, hg), (C_RG, rg), (C_RW, rw), (C_X2, x2), (C_Y2, y2),
                 (C_RA, ra), (C_ST, score_t), (C_SR, score_r), (C_IDX, flat_f),
                 (14, jnp.zeros((ROWS, LANES), f32)),
                 (15, jnp.zeros((ROWS, LANES), f32))):
        table_ref[c] = v

    li_r = lax.broadcasted_iota(i32, (LANES, LANES), 0)
    li_c = lax.broadcasted_iota(i32, (LANES, LANES), 1)
    l_incl = (li_r <= li_c).astype(f32)
    rr = lax.broadcasted_iota(i32, (ROWS, ROWS), 0)
    rc = lax.broadcasted_iota(i32, (ROWS, ROWS), 1)
    l_strict = (rc < rr).astype(f32)

    key_t = _sortkey(score_t, valid)
    key_r = _sortkey(score_r, valid)
    tt, tr = _kth_key2(key_t, key_r, K)
    destt_ref[...] = _dest_of(key_t, tt, l_incl, l_strict, KPAD)
    destr_ref[...] = _dest_of(key_r, tr, l_incl, l_strict,
                              KPAD) + i32(R_OFF)


def _prep(resid, logit, res, reg, anc):
    return pl.pallas_call(
        _prep_body,
        out_shape=[
            jax.ShapeDtypeStruct((NCH, ROWS, LANES), f32),
            jax.ShapeDtypeStruct((ROWS, LANES), i32),
            jax.ShapeDtypeStruct((ROWS, LANES), i32),
        ],
    )(resid, logit, res, reg, anc)


def _scatter_rows(table2d, destt, destr):
    mesh = plsc.VectorSubcoreMesh(core_axis_name="c", subcore_axis_name="s")
    nchunk = EPW // LANES

    @functools.partial(
        pl.kernel, mesh=mesh,
        out_type=jax.ShapeDtypeStruct((OUT_ROWS, NCH), f32),
        compiler_params=pltpu.CompilerParams(use_tc_tiling_on_sc=False),
        scratch_types=[
            pltpu.VMEM((EPW, NCH), f32),
            pltpu.VMEM((nchunk, LANES), i32),
            pltpu.VMEM((nchunk, LANES), i32),
            pltpu.SemaphoreType.DMA,
        ],
    )
    def k(table_hbm, dt_hbm, dr_hbm, out_hbm, rows_v, dt_v, dr_v, sem):
        wid = lax.axis_index("s") * 2 + lax.axis_index("c")
        base = wid * EPW
        pltpu.sync_copy(table_hbm.at[pl.ds(base, EPW)], rows_v)
        pltpu.sync_copy(dt_hbm.at[wid], dt_v)
        pltpu.sync_copy(dr_hbm.at[wid], dr_v)
        copies = []
        for c in range(nchunk):
            rows_c = rows_v.at[pl.ds(c * LANES, LANES)]
            copies.append(pltpu.async_copy(rows_c, out_hbm.at[dt_v.at[c]],
                                           sem))
            copies.append(pltpu.async_copy(rows_c, out_hbm.at[dr_v.at[c]],
                                           sem))
        for cp in copies:
            cp.wait()

    return k(table2d, destt, destr)


def _row(t, c):
    return t[c:c + 1, :]


def _iou_prec_block(colT, rowpre, blk):
    x1j, x2j, y1j, y2j, areaj, sj, ij = rowpre
    sl = slice(blk * BLK, (blk + 1) * BLK)
    xi = colT[sl, 0:1]
    yi = colT[sl, 1:2]
    wi = colT[sl, 2:3]
    li = colT[sl, 3:4]
    si = colT[sl, 4:5]
    ii = colT[sl, 5:6]
    x1i = xi - wi * 0.5
    x2i = xi + wi * 0.5
    y1i = yi - li * 0.5
    y2i = yi + li * 0.5
    areai = (x2i - x1i) * (y2i - y1i)
    ix1 = jnp.maximum(x1i, x1j)
    iy1 = jnp.maximum(y1i, y1j)
    ix2 = jnp.minimum(x2i, x2j)
    iy2 = jnp.minimum(y2i, y2j)
    inter = jnp.clip(ix2 - ix1, 0.0) * jnp.clip(iy2 - iy1, 0.0)
    union = areai + areaj - inter
    iou = inter / jnp.maximum(union, 1e-8)
    prec = (si > sj) | ((si == sj) & (ii < ij))
    vi = (lax.broadcasted_iota(i32, (BLK, 1), 0) + blk * BLK) < K
    return ((iou > 0.01) & prec & vi).astype(f32)


def _nms_fixpoint(M_ref, keep_ref):
    keep_ref[...] = jnp.ones((1, KPAD), f32)

    def cond(c):
        return c > 0

    def body(c):
        kv = keep_ref[...]
        supp = jnp.dot(kv, M_ref[...], preferred_element_type=f32)
        knew = (supp < 0.5).astype(f32)
        changed = jnp.sum(jnp.abs(knew - kv))
        keep_ref[...] = knew
        return (changed > 0.0).astype(i32)

    lax.while_loop(cond, body, i32(1))
    return keep_ref[...]


def _rank_of(colsub, rowsub, k_row, valid_row):
    s_row, i_row = rowsub
    m_row = jnp.where((k_row > 0.5) & valid_row, s_row, NEG)
    rank = jnp.zeros((1, KPAD), f32)
    for blk in range(KPAD // BLK):
        sl = slice(blk * BLK, (blk + 1) * BLK)
        eye = (lax.broadcasted_iota(i32, (BLK, BLK), 0)
               == lax.broadcasted_iota(i32, (BLK, BLK), 1))
        k_col = jnp.sum(jnp.where(eye, k_row[:, sl], 0.0), axis=1,
                        keepdims=True)
        v_col = (lax.broadcasted_iota(i32, (BLK, 1), 0) + blk * BLK) < K
        s_col = colsub[sl, 0:1]
        i_col = colsub[sl, 1:2]
        m_col = jnp.where((k_col > 0.5) & v_col, s_col, NEG)
        gt = (m_col > m_row) | ((m_col == m_row) & (i_col < i_row))
        rank = rank + jnp.sum(gt.astype(f32), axis=0, keepdims=True)
    return rank, m_row


def _emit(out_ref, p, rowT, chans, rank, mask_row):
    sel = (lax.broadcasted_iota(i32, (POST + 28, KPAD), 0).astype(f32)
           == rank).astype(f32)
    cols = []
    for c in chans:
        d = jnp.where(mask_row > 0.5, _row(rowT, c), 0.0)
        cols.append(jnp.sum(sel * d, axis=1, keepdims=True))
    out_ref[p] = jnp.concatenate(cols, axis=1)


def _nms_body(tc_ref, tr_ref, rc_ref, rr_ref, out_ref, M_ref, keep_ref):
    valid_row = lax.broadcasted_iota(i32, (1, KPAD), 1) < K

    def run(colT_full, rowT, xch, ych, sch, outs):
        colsubM = jnp.concatenate(
            [colT_full[:, xch:xch + 1], colT_full[:, ych:ych + 1],
             colT_full[:, C_WG:C_WG + 1], colT_full[:, C_LG:C_LG + 1],
             colT_full[:, sch:sch + 1], colT_full[:, C_IDX:C_IDX + 1]],
            axis=1)
        xj = _row(rowT, xch)
        yj = _row(rowT, ych)
        wj = _row(rowT, C_WG)
        lj = _row(rowT, C_LG)
        sj = _row(rowT, sch)
        ij = _row(rowT, C_IDX)
        x1j = xj - wj * 0.5
        x2j = xj + wj * 0.5
        y1j = yj - lj * 0.5
        y2j = yj + lj * 0.5
        areaj = (x2j - x1j) * (y2j - y1j)
        rowpre = (x1j, x2j, y1j, y2j, areaj, sj, ij)
        for blk in range(KPAD // BLK):
            sl = slice(blk * BLK, (blk + 1) * BLK)
            M_ref[sl, :] = _iou_prec_block(colsubM, rowpre, blk)
        k_row = _nms_fixpoint(M_ref, keep_ref)
        colsubR = jnp.concatenate(
            [colT_full[:, sch:sch + 1], colT_full[:, C_IDX:C_IDX + 1]], axis=1)
        rank, _ = _rank_of(colsubR, (sj, ij), k_row, valid_row)
        zj = _row(rowT, C_ZG)
        in_rng = ((xj >= 0.0) & (xj <= 70.4) & (yj >= -40.0) & (yj <= 40.0)
                  & (zj >= -2.2) & (zj <= 0.8))
        mask_row = (in_rng & (k_row > 0.5) & valid_row).astype(f32)
        for p, ang in outs:
            _emit(out_ref, p, rowT,
                  (xch, ych, C_ZG, C_WG, C_LG, C_HG, ang, sch), rank, mask_row)

    tc = tc_ref[...]
    rc = rc_ref[...]
    run(tc, tr_ref[...], C_XG, C_YG, C_ST, ((0, C_RG), (2, C_RW)))
    run(tc, tr_ref[...], C_X2, C_Y2, C_ST, ((1, C_RA),))
    run(rc, rr_ref[...], C_XG, C_YG, C_SR, ((3, C_RW),))


def _nms(tc, tr, rc, rr):
    return pl.pallas_call(
        _nms_body,
        out_shape=jax.ShapeDtypeStruct((4, POST + 28, 8), f32),
        scratch_shapes=[
            pltpu.VMEM((KPAD, KPAD), f32),
            pltpu.VMEM((1, KPAD), f32),
        ],
    )(tc, tr, rc, rr)


def kernel(class_logits, box_regression, anchors, res_scores,
           post_max_into_pre_max):
    resid = (jnp.asarray(post_max_into_pre_max, f32) - 2000.0).reshape(1, 1)
    pad = NPAD - N
    logit = jnp.pad(class_logits[:, 0], (0, pad)).reshape(ROWS, LANES)
    res = jnp.pad(res_scores, (0, pad)).reshape(ROWS, LANES)
    reg = jnp.pad(box_regression, ((0, pad), (0, 0))).T.reshape(7, ROWS, LANES)
    anc = jnp.pad(anchors, ((0, pad), (0, 0))).T.reshape(7, ROWS, LANES)

    table, destt, destr = _prep(resid, logit, res, reg, anc)
    table2d = table.reshape(NCH, NPAD).T
    rows = _scatter_rows(table2d,
                         destt.reshape(NW, EPW // LANES, LANES),
                         destr.reshape(NW, EPW // LANES, LANES))

    tcol = rows[:KPAD]
    rcol = rows[R_OFF:R_OFF + KPAD]
    out = _nms(tcol, tcol.T, rcol, rcol.T)
    return out[:, :POST, :]


# speedup vs baseline: 1.4032x; 1.4032x over previous
import functools

import jax
import jax.numpy as jnp
from jax import lax
from jax.experimental import pallas as pl
from jax.experimental.pallas import tpu as pltpu
from jax.experimental.pallas import tpu_sc as plsc

N = 20000
LANES = 128
ROWS = 160
NPAD = ROWS * LANES
K = 2000
KPAD = 2048
BLK = 256
NCH = 16
POST = 100
NEG = -3.0e38
NW = 32
EPW = NPAD // NW
R_OFF = KPAD + 8
DUMP_T = R_OFF + KPAD + 8
DUMP_R = DUMP_T + NPAD
OUT_ROWS = DUMP_R + NPAD

f32 = jnp.float32
i32 = jnp.int32
u32 = jnp.uint32

C_XG, C_YG, C_ZG, C_WG, C_LG, C_HG, C_RG, C_RW = 0, 1, 2, 3, 4, 5, 6, 7
C_X2, C_Y2, C_RA, C_ST, C_SR, C_IDX = 8, 9, 10, 11, 12, 13


def _sortkey(s, valid):
    u = lax.bitcast_convert_type(s, u32)
    key = jnp.where(u >= u32(0x80000000), ~u, u | u32(0x80000000))
    return jnp.where(valid, key, u32(0))


def _kth_key2(key_a, key_b, kwant):
    ta = u32(0)
    tb = u32(0)
    for b in range(31, -1, -1):
        ca = ta | u32(1 << b)
        cb = tb | u32(1 << b)
        cnta = jnp.sum((key_a >= ca).astype(i32))
        cntb = jnp.sum((key_b >= cb).astype(i32))
        ta = jnp.where(cnta >= kwant, ca, ta)
        tb = jnp.where(cntb >= kwant, cb, tb)
    return ta, tb


def _excl_cumsum(x, l_incl, l_strict_rows):
    incl = jnp.dot(x, l_incl, preferred_element_type=f32)
    rowtot = jnp.sum(x, axis=1, keepdims=True)
    prev_rows = jnp.dot(l_strict_rows, rowtot, preferred_element_type=f32)
    return incl - x + prev_rows


def _dest_of(key, t, l_incl, l_strict_rows, base, dump, flat):
    gt = key > t
    eq = key == t
    c_gt = jnp.sum(gt.astype(f32))
    need = f32(K) - c_gt
    tie_rank = _excl_cumsum(eq.astype(f32), l_incl, l_strict_rows)
    sel = gt | (eq & (tie_rank < need))
    pos = _excl_cumsum(sel.astype(f32), l_incl, l_strict_rows).astype(i32)
    return jnp.where(sel, pos + i32(base), flat + i32(dump))


def _prep_body(resid_ref, logit_ref, res_ref, reg_ref, anc_ref,
               table_ref, destt_ref, destr_ref):
    resid = resid_ref[0, 0]
    logit = logit_ref[...]
    res = res_ref[...]
    xa, ya, za = anc_ref[0], anc_ref[1], anc_ref[2]
    wa, la, ha, ra = anc_ref[3], anc_ref[4], anc_ref[5], anc_ref[6]
    r0, r1, r2 = reg_ref[0], reg_ref[1], reg_ref[2]
    r3, r4, r5, r6 = reg_ref[3], reg_ref[4], reg_ref[5], reg_ref[6]

    diag = jnp.sqrt(la * la + wa * wa)
    xg = r0 / 10.0 * diag + xa
    yg = r1 / 10.0 * diag + ya
    zg = r2 / 10.0 * ha + za
    wg = jnp.exp(r3 / 5.0) * wa
    lg = jnp.exp(r4 / 5.0) * la
    hg = jnp.exp(r5 / 5.0) * ha
    rg = r6 / 10.0 + ra
    rw = jnp.arctan2(jnp.sin(rg), jnp.cos(rg))
    x2 = r0 / 10.0 * wa + xa
    y2 = r1 / 10.0 * la + ya

    score_t = jax.nn.sigmoid(jax.nn.sigmoid(logit)) + resid
    score_r = res + resid

    row_i = lax.broadcasted_iota(i32, (ROWS, LANES), 0)
    lane_i = lax.broadcasted_iota(i32, (ROWS, LANES), 1)
    flat = row_i * LANES + lane_i
    valid = flat < N
    flat_f = flat.astype(f32)

    for c, v in ((C_XG, xg), (C_YG, yg), (C_ZG, zg), (C_WG, wg), (C_LG, lg),
                 (C_HG, hg), (C_RG, rg), (C_RW, rw), (C_X2, x2), (C_Y2, y2),
                 (C_RA, ra), (C_ST, score_t), (C_SR, score_r), (C_IDX, flat_f),
                 (14, jnp.zeros((ROWS, LANES), f32)),
                 (15, jnp.zeros((ROWS, LANES), f32))):
        table_ref[c] = v

    li_r = lax.broadcasted_iota(i32, (LANES, LANES), 0)
    li_c = lax.broadcasted_iota(i32, (LANES, LANES), 1)
    l_incl = (li_r <= li_c).astype(f32)
    rr = lax.broadcasted_iota(i32, (ROWS, ROWS), 0)
    rc = lax.broadcasted_iota(i32, (ROWS, ROWS), 1)
    l_strict = (rc < rr).astype(f32)

    key_t = _sortkey(score_t, valid)
    key_r = _sortkey(score_r, valid)
    tt, tr = _kth_key2(key_t, key_r, K)
    destt_ref[...] = _dest_of(key_t, tt, l_incl, l_strict, 0, DUMP_T, flat)
    destr_ref[...] = _dest_of(key_r, tr, l_incl, l_strict, R_OFF, DUMP_R,
                              flat)


def _prep(resid, logit, res, reg, anc):
    return pl.pallas_call(
        _prep_body,
        out_shape=[
            jax.ShapeDtypeStruct((NCH, ROWS, LANES), f32),
            jax.ShapeDtypeStruct((ROWS, LANES), i32),
            jax.ShapeDtypeStruct((ROWS, LANES), i32),
        ],
    )(resid, logit, res, reg, anc)


def _scatter_rows(table2d, destt, destr):
    mesh = plsc.VectorSubcoreMesh(core_axis_name="c", subcore_axis_name="s")
    nchunk = EPW // LANES

    @functools.partial(
        pl.kernel, mesh=mesh,
        out_type=jax.ShapeDtypeStruct((OUT_ROWS, NCH), f32),
        compiler_params=pltpu.CompilerParams(use_tc_tiling_on_sc=False),
        scratch_types=[
            pltpu.VMEM((EPW, NCH), f32),
            pltpu.VMEM((nchunk, LANES), i32),
            pltpu.VMEM((nchunk, LANES), i32),
            pltpu.SemaphoreType.DMA,
        ],
    )
    def k(table_hbm, dt_hbm, dr_hbm, out_hbm, rows_v, dt_v, dr_v, sem):
        wid = lax.axis_index("s") * 2 + lax.axis_index("c")
        base = wid * EPW
        pltpu.sync_copy(table_hbm.at[pl.ds(base, EPW)], rows_v)
        pltpu.sync_copy(dt_hbm.at[wid], dt_v)
        pltpu.sync_copy(dr_hbm.at[wid], dr_v)
        copies = []
        for c in range(nchunk):
            rows_c = rows_v.at[pl.ds(c * LANES, LANES)]
            copies.append(pltpu.async_copy(rows_c, out_hbm.at[dt_v.at[c]],
                                           sem))
            copies.append(pltpu.async_copy(rows_c, out_hbm.at[dr_v.at[c]],
                                           sem))
        for cp in copies:
            cp.wait()

    return k(table2d, destt, destr)


def _row(t, c):
    return t[c:c + 1, :]


def _iou_prec_block(colT, rowpre, blk):
    x1j, x2j, y1j, y2j, areaj, sj, ij = rowpre
    sl = slice(blk * BLK, (blk + 1) * BLK)
    xi = colT[sl, 0:1]
    yi = colT[sl, 1:2]
    wi = colT[sl, 2:3]
    li = colT[sl, 3:4]
    si = colT[sl, 4:5]
    ii = colT[sl, 5:6]
    x1i = xi - wi * 0.5
    x2i = xi + wi * 0.5
    y1i = yi - li * 0.5
    y2i = yi + li * 0.5
    areai = (x2i - x1i) * (y2i - y1i)
    ix1 = jnp.maximum(x1i, x1j)
    iy1 = jnp.maximum(y1i, y1j)
    ix2 = jnp.minimum(x2i, x2j)
    iy2 = jnp.minimum(y2i, y2j)
    inter = jnp.clip(ix2 - ix1, 0.0) * jnp.clip(iy2 - iy1, 0.0)
    union = areai + areaj - inter
    iou = inter / jnp.maximum(union, 1e-8)
    prec = (si > sj) | ((si == sj) & (ii < ij))
    vi = (lax.broadcasted_iota(i32, (BLK, 1), 0) + blk * BLK) < K
    return ((iou > 0.01) & prec & vi).astype(f32)


def _nms_fixpoint(M_ref, keep_ref):
    keep_ref[...] = jnp.ones((1, KPAD), f32)

    def cond(c):
        return c > 0

    def body(c):
        kv = keep_ref[...]
        supp = jnp.dot(kv, M_ref[...], preferred_element_type=f32)
        knew = (supp < 0.5).astype(f32)
        changed = jnp.sum(jnp.abs(knew - kv))
        keep_ref[...] = knew
        return (changed > 0.0).astype(i32)

    lax.while_loop(cond, body, i32(1))
    return keep_ref[...]


def _rank_of(colsub, rowsub, k_row, valid_row):
    s_row, i_row = rowsub
    m_row = jnp.where((k_row > 0.5) & valid_row, s_row, NEG)
    rank = jnp.zeros((1, KPAD), f32)
    for blk in range(KPAD // BLK):
        sl = slice(blk * BLK, (blk + 1) * BLK)
        eye = (lax.broadcasted_iota(i32, (BLK, BLK), 0)
               == lax.broadcasted_iota(i32, (BLK, BLK), 1))
        k_col = jnp.sum(jnp.where(eye, k_row[:, sl], 0.0), axis=1,
                        keepdims=True)
        v_col = (lax.broadcasted_iota(i32, (BLK, 1), 0) + blk * BLK) < K
        s_col = colsub[sl, 0:1]
        i_col = colsub[sl, 1:2]
        m_col = jnp.where((k_col > 0.5) & v_col, s_col, NEG)
        gt = (m_col > m_row) | ((m_col == m_row) & (i_col < i_row))
        rank = rank + jnp.sum(gt.astype(f32), axis=0, keepdims=True)
    return rank, m_row


def _emit(out_ref, p, rowT, chans, rank, mask_row):
    sel = (lax.broadcasted_iota(i32, (POST + 28, KPAD), 0).astype(f32)
           == rank).astype(f32)
    cols = []
    for c in chans:
        d = jnp.where(mask_row > 0.5, _row(rowT, c), 0.0)
        cols.append(jnp.sum(sel * d, axis=1, keepdims=True))
    out_ref[p] = jnp.concatenate(cols, axis=1)


def _nms_body(tc_ref, tr_ref, rc_ref, rr_ref, out_ref, M_ref, keep_ref):
    valid_row = lax.broadcasted_iota(i32, (1, KPAD), 1) < K

    def run(colT_full, rowT, xch, ych, sch, outs):
        colsubM = jnp.concatenate(
            [colT_full[:, xch:xch + 1], colT_full[:, ych:ych + 1],
             colT_full[:, C_WG:C_WG + 1], colT_full[:, C_LG:C_LG + 1],
             colT_full[:, sch:sch + 1], colT_full[:, C_IDX:C_IDX + 1]],
            axis=1)
        xj = _row(rowT, xch)
        yj = _row(rowT, ych)
        wj = _row(rowT, C_WG)
        lj = _row(rowT, C_LG)
        sj = _row(rowT, sch)
        ij = _row(rowT, C_IDX)
        x1j = xj - wj * 0.5
        x2j = xj + wj * 0.5
        y1j = yj - lj * 0.5
        y2j = yj + lj * 0.5
        areaj = (x2j - x1j) * (y2j - y1j)
        rowpre = (x1j, x2j, y1j, y2j, areaj, sj, ij)
        for blk in range(KPAD // BLK):
            sl = slice(blk * BLK, (blk + 1) * BLK)
            M_ref[sl, :] = _iou_prec_block(colsubM, rowpre, blk)
        k_row = _nms_fixpoint(M_ref, keep_ref)
        colsubR = jnp.concatenate(
            [colT_full[:, sch:sch + 1], colT_full[:, C_IDX:C_IDX + 1]], axis=1)
        rank, _ = _rank_of(colsubR, (sj, ij), k_row, valid_row)
        zj = _row(rowT, C_ZG)
        in_rng = ((xj >= 0.0) & (xj <= 70.4) & (yj >= -40.0) & (yj <= 40.0)
                  & (zj >= -2.2) & (zj <= 0.8))
        mask_row = (in_rng & (k_row > 0.5) & valid_row).astype(f32)
        for p, ang in outs:
            _emit(out_ref, p, rowT,
                  (xch, ych, C_ZG, C_WG, C_LG, C_HG, ang, sch), rank, mask_row)

    tc = tc_ref[...]
    rc = rc_ref[...]
    run(tc, tr_ref[...], C_XG, C_YG, C_ST, ((0, C_RG), (2, C_RW)))
    run(tc, tr_ref[...], C_X2, C_Y2, C_ST, ((1, C_RA),))
    run(rc, rr_ref[...], C_XG, C_YG, C_SR, ((3, C_RW),))


def _nms(tc, tr, rc, rr):
    return pl.pallas_call(
        _nms_body,
        out_shape=jax.ShapeDtypeStruct((4, POST + 28, 8), f32),
        scratch_shapes=[
            pltpu.VMEM((KPAD, KPAD), f32),
            pltpu.VMEM((1, KPAD), f32),
        ],
    )(tc, tr, rc, rr)


def kernel(class_logits, box_regression, anchors, res_scores,
           post_max_into_pre_max):
    resid = (jnp.asarray(post_max_into_pre_max, f32) - 2000.0).reshape(1, 1)
    pad = NPAD - N
    logit = jnp.pad(class_logits[:, 0], (0, pad)).reshape(ROWS, LANES)
    res = jnp.pad(res_scores, (0, pad)).reshape(ROWS, LANES)
    reg = jnp.pad(box_regression, ((0, pad), (0, 0))).T.reshape(7, ROWS, LANES)
    anc = jnp.pad(anchors, ((0, pad), (0, 0))).T.reshape(7, ROWS, LANES)

    table, destt, destr = _prep(resid, logit, res, reg, anc)
    table2d = table.reshape(NCH, NPAD).T
    rows = _scatter_rows(table2d,
                         destt.reshape(NW, EPW // LANES, LANES),
                         destr.reshape(NW, EPW // LANES, LANES))

    tcol = rows[:KPAD]
    rcol = rows[R_OFF:R_OFF + KPAD]
    out = _nms(tcol, tcol.T, rcol, rcol.T)
    return out[:, :POST, :]


# speedup vs baseline: 1.4101x; 1.0049x over previous
import functools

import jax
import jax.numpy as jnp
from jax import lax
from jax.experimental import pallas as pl
from jax.experimental.pallas import tpu as pltpu
from jax.experimental.pallas import tpu_sc as plsc

N = 20000
LANES = 128
ROWS = 160
NPAD = ROWS * LANES
K = 2000
KPAD = 2048
BLK = 256
NCH = 16
POST = 100
NEG = -3.0e38
NW = 32
EPW = NPAD // NW
R_OFF = KPAD + 8
DUMP_T = R_OFF + KPAD + 8
DUMP_R = DUMP_T + NPAD
OUT_ROWS = DUMP_R + NPAD

f32 = jnp.float32
i32 = jnp.int32
u32 = jnp.uint32

C_XG, C_YG, C_ZG, C_WG, C_LG, C_HG, C_RG, C_RW = 0, 1, 2, 3, 4, 5, 6, 7
C_X2, C_Y2, C_RA, C_ST, C_SR, C_IDX = 8, 9, 10, 11, 12, 13


def _sortkey(s, valid):
    u = lax.bitcast_convert_type(s, u32)
    key = jnp.where(u >= u32(0x80000000), ~u, u | u32(0x80000000))
    return jnp.where(valid, key, u32(0))


def _kth_key2(key_a, key_b, kwant):
    ta = u32(0)
    tb = u32(0)
    for b in range(31, -1, -1):
        ca = ta | u32(1 << b)
        cb = tb | u32(1 << b)
        cnta = jnp.sum((key_a >= ca).astype(i32))
        cntb = jnp.sum((key_b >= cb).astype(i32))
        ta = jnp.where(cnta >= kwant, ca, ta)
        tb = jnp.where(cntb >= kwant, cb, tb)
    return ta, tb


def _excl_cumsum(x, l_incl, l_strict_rows):
    incl = jnp.dot(x, l_incl, preferred_element_type=f32)
    rowtot = jnp.sum(x, axis=1, keepdims=True)
    prev_rows = jnp.dot(l_strict_rows, rowtot, preferred_element_type=f32)
    return incl - x + prev_rows


def _dest_of(key, t, l_incl, l_strict_rows, base, dump, flat):
    gt = key > t
    eq = key == t
    c_gt = jnp.sum(gt.astype(f32))
    need = f32(K) - c_gt
    tie_rank = _excl_cumsum(eq.astype(f32), l_incl, l_strict_rows)
    sel = gt | (eq & (tie_rank < need))
    pos = _excl_cumsum(sel.astype(f32), l_incl, l_strict_rows).astype(i32)
    return jnp.where(sel, pos + i32(base), flat + i32(dump))


def _prep_body(resid_ref, logit_ref, res_ref, reg_ref, anc_ref,
               table_ref, destt_ref, destr_ref):
    resid = resid_ref[0, 0]
    logit = logit_ref[...]
    res = res_ref[...]
    xa, ya, za = anc_ref[0], anc_ref[1], anc_ref[2]
    wa, la, ha, ra = anc_ref[3], anc_ref[4], anc_ref[5], anc_ref[6]
    r0, r1, r2 = reg_ref[0], reg_ref[1], reg_ref[2]
    r3, r4, r5, r6 = reg_ref[3], reg_ref[4], reg_ref[5], reg_ref[6]

    diag = jnp.sqrt(la * la + wa * wa)
    xg = r0 / 10.0 * diag + xa
    yg = r1 / 10.0 * diag + ya
    zg = r2 / 10.0 * ha + za
    wg = jnp.exp(r3 / 5.0) * wa
    lg = jnp.exp(r4 / 5.0) * la
    hg = jnp.exp(r5 / 5.0) * ha
    rg = r6 / 10.0 + ra
    rw = jnp.arctan2(jnp.sin(rg), jnp.cos(rg))
    x2 = r0 / 10.0 * wa + xa
    y2 = r1 / 10.0 * la + ya

    score_t = jax.nn.sigmoid(jax.nn.sigmoid(logit)) + resid
    score_r = res + resid

    row_i = lax.broadcasted_iota(i32, (ROWS, LANES), 0)
    lane_i = lax.broadcasted_iota(i32, (ROWS, LANES), 1)
    flat = row_i * LANES + lane_i
    valid = flat < N
    flat_f = flat.astype(f32)

    for c, v in ((C_XG, xg), (C_YG, yg), (C_ZG, zg), (C_WG, wg), (C_LG, lg),
                 (C_HG, hg), (C_RG, rg), (C_RW, rw), (C_X2, x2), (C_Y2, y2),
                 (C_RA, ra), (C_ST, score_t), (C_SR, score_r), (C_IDX, flat_f),
                 (14, jnp.zeros((ROWS, LANES), f32)),
                 (15, jnp.zeros((ROWS, LANES), f32))):
        table_ref[c] = v

    li_r = lax.broadcasted_iota(i32, (LANES, LANES), 0)
    li_c = lax.broadcasted_iota(i32, (LANES, LANES), 1)
    l_incl = (li_r <= li_c).astype(f32)
    rr = lax.broadcasted_iota(i32, (ROWS, ROWS), 0)
    rc = lax.broadcasted_iota(i32, (ROWS, ROWS), 1)
    l_strict = (rc < rr).astype(f32)

    key_t = _sortkey(score_t, valid)
    key_r = _sortkey(score_r, valid)
    tt, tr = _kth_key2(key_t, key_r, K)
    destt_ref[...] = _dest_of(key_t, tt, l_incl, l_strict, 0, DUMP_T, flat)
    destr_ref[...] = _dest_of(key_r, tr, l_incl, l_strict, R_OFF, DUMP_R,
                              flat)


def _prep(resid, logit, res, reg, anc):
    return pl.pallas_call(
        _prep_body,
        out_shape=[
            jax.ShapeDtypeStruct((NCH, ROWS, LANES), f32),
            jax.ShapeDtypeStruct((ROWS, LANES), i32),
            jax.ShapeDtypeStruct((ROWS, LANES), i32),
        ],
    )(resid, logit, res, reg, anc)


def _scatter_rows(table2d, destt, destr):
    mesh = plsc.VectorSubcoreMesh(core_axis_name="c", subcore_axis_name="s")
    nchunk = EPW // LANES

    @functools.partial(
        pl.kernel, mesh=mesh,
        out_type=jax.ShapeDtypeStruct((OUT_ROWS, NCH), f32),
        compiler_params=pltpu.CompilerParams(use_tc_tiling_on_sc=False),
        scratch_types=[
            pltpu.VMEM((EPW, NCH), f32),
            pltpu.VMEM((nchunk, LANES), i32),
            pltpu.VMEM((nchunk, LANES), i32),
            pltpu.SemaphoreType.DMA,
        ],
    )
    def k(table_hbm, dt_hbm, dr_hbm, out_hbm, rows_v, dt_v, dr_v, sem):
        wid = lax.axis_index("s") * 2 + lax.axis_index("c")
        base = wid * EPW
        pltpu.sync_copy(table_hbm.at[pl.ds(base, EPW)], rows_v)
        pltpu.sync_copy(dt_hbm.at[wid], dt_v)
        pltpu.sync_copy(dr_hbm.at[wid], dr_v)
        copies = []
        for c in range(nchunk):
            rows_c = rows_v.at[pl.ds(c * LANES, LANES)]
            copies.append(pltpu.async_copy(rows_c, out_hbm.at[dt_v.at[c]],
                                           sem))
            copies.append(pltpu.async_copy(rows_c, out_hbm.at[dr_v.at[c]],
                                           sem))
        for cp in copies:
            cp.wait()

    return k(table2d, destt, destr)


def _row(t, c):
    return t[c:c + 1, :]


def _iou_prec_block(colT, rowpre, blk):
    x1j, x2j, y1j, y2j, areaj, sj, ij = rowpre
    sl = slice(blk * BLK, (blk + 1) * BLK)
    xi = colT[sl, 0:1]
    yi = colT[sl, 1:2]
    wi = colT[sl, 2:3]
    li = colT[sl, 3:4]
    si = colT[sl, 4:5]
    ii = colT[sl, 5:6]
    x1i = xi - wi * 0.5
    x2i = xi + wi * 0.5
    y1i = yi - li * 0.5
    y2i = yi + li * 0.5
    areai = (x2i - x1i) * (y2i - y1i)
    ix1 = jnp.maximum(x1i, x1j)
    iy1 = jnp.maximum(y1i, y1j)
    ix2 = jnp.minimum(x2i, x2j)
    iy2 = jnp.minimum(y2i, y2j)
    inter = jnp.clip(ix2 - ix1, 0.0) * jnp.clip(iy2 - iy1, 0.0)
    union = areai + areaj - inter
    iou = inter / jnp.maximum(union, 1e-8)
    prec = (si > sj) | ((si == sj) & (ii < ij))
    vi = (lax.broadcasted_iota(i32, (BLK, 1), 0) + blk * BLK) < K
    return ((iou > 0.01) & prec & vi).astype(jnp.bfloat16)


def _nms_fixpoint(M_ref, keep_ref):
    keep_ref[...] = jnp.ones((1, KPAD), jnp.bfloat16)

    def cond(c):
        return c > 0

    def body(c):
        kv = keep_ref[...]
        supp = jnp.dot(kv, M_ref[...], preferred_element_type=f32)
        knew = (supp < 0.5).astype(jnp.bfloat16)
        changed = jnp.sum(jnp.abs((knew - kv).astype(f32)))
        keep_ref[...] = knew
        return (changed > 0.0).astype(i32)

    lax.while_loop(cond, body, i32(1))
    return keep_ref[...].astype(f32)


def _rank_of(colsub, rowsub, k_row, valid_row):
    s_row, i_row = rowsub
    m_row = jnp.where((k_row > 0.5) & valid_row, s_row, NEG)
    rank = jnp.zeros((1, KPAD), f32)
    for blk in range(KPAD // BLK):
        sl = slice(blk * BLK, (blk + 1) * BLK)
        eye = (lax.broadcasted_iota(i32, (BLK, BLK), 0)
               == lax.broadcasted_iota(i32, (BLK, BLK), 1))
        k_col = jnp.sum(jnp.where(eye, k_row[:, sl], 0.0), axis=1,
                        keepdims=True)
        v_col = (lax.broadcasted_iota(i32, (BLK, 1), 0) + blk * BLK) < K
        s_col = colsub[sl, 0:1]
        i_col = colsub[sl, 1:2]
        m_col = jnp.where((k_col > 0.5) & v_col, s_col, NEG)
        gt = (m_col > m_row) | ((m_col == m_row) & (i_col < i_row))
        rank = rank + jnp.sum(gt.astype(f32), axis=0, keepdims=True)
    return rank, m_row


def _emit(out_ref, p, rowT, chans, rank, mask_row):
    sel = (lax.broadcasted_iota(i32, (POST + 28, KPAD), 0).astype(f32)
           == rank).astype(f32)
    cols = []
    for c in chans:
        d = jnp.where(mask_row > 0.5, _row(rowT, c), 0.0)
        cols.append(jnp.sum(sel * d, axis=1, keepdims=True))
    out_ref[p] = jnp.concatenate(cols, axis=1)


def _nms_body(tc_ref, tr_ref, rc_ref, rr_ref, out_ref, M_ref, keep_ref):
    valid_row = lax.broadcasted_iota(i32, (1, KPAD), 1) < K

    def run(colT_full, rowT, xch, ych, sch, outs):
        colsubM = jnp.concatenate(
            [colT_full[:, xch:xch + 1], colT_full[:, ych:ych + 1],
             colT_full[:, C_WG:C_WG + 1], colT_full[:, C_LG:C_LG + 1],
             colT_full[:, sch:sch + 1], colT_full[:, C_IDX:C_IDX + 1]],
            axis=1)
        xj = _row(rowT, xch)
        yj = _row(rowT, ych)
        wj = _row(rowT, C_WG)
        lj = _row(rowT, C_LG)
        sj = _row(rowT, sch)
        ij = _row(rowT, C_IDX)
        x1j = xj - wj * 0.5
        x2j = xj + wj * 0.5
        y1j = yj - lj * 0.5
        y2j = yj + lj * 0.5
        areaj = (x2j - x1j) * (y2j - y1j)
        rowpre = (x1j, x2j, y1j, y2j, areaj, sj, ij)
        for blk in range(KPAD // BLK):
            sl = slice(blk * BLK, (blk + 1) * BLK)
            M_ref[sl, :] = _iou_prec_block(colsubM, rowpre, blk)
        k_row = _nms_fixpoint(M_ref, keep_ref)
        colsubR = jnp.concatenate(
            [colT_full[:, sch:sch + 1], colT_full[:, C_IDX:C_IDX + 1]], axis=1)
        rank, _ = _rank_of(colsubR, (sj, ij), k_row, valid_row)
        zj = _row(rowT, C_ZG)
        in_rng = ((xj >= 0.0) & (xj <= 70.4) & (yj >= -40.0) & (yj <= 40.0)
                  & (zj >= -2.2) & (zj <= 0.8))
        mask_row = (in_rng & (k_row > 0.5) & valid_row).astype(f32)
        for p, ang in outs:
            _emit(out_ref, p, rowT,
                  (xch, ych, C_ZG, C_WG, C_LG, C_HG, ang, sch), rank, mask_row)

    tc = tc_ref[...]
    rc = rc_ref[...]
    run(tc, tr_ref[...], C_XG, C_YG, C_ST, ((0, C_RG), (2, C_RW)))
    run(tc, tr_ref[...], C_X2, C_Y2, C_ST, ((1, C_RA),))
    run(rc, rr_ref[...], C_XG, C_YG, C_SR, ((3, C_RW),))


def _nms(tc, tr, rc, rr):
    return pl.pallas_call(
        _nms_body,
        out_shape=jax.ShapeDtypeStruct((4, POST + 28, 8), f32),
        scratch_shapes=[
            pltpu.VMEM((KPAD, KPAD), jnp.bfloat16),
            pltpu.VMEM((1, KPAD), jnp.bfloat16),
        ],
    )(tc, tr, rc, rr)


def kernel(class_logits, box_regression, anchors, res_scores,
           post_max_into_pre_max):
    resid = (jnp.asarray(post_max_into_pre_max, f32) - 2000.0).reshape(1, 1)
    pad = NPAD - N
    logit = jnp.pad(class_logits[:, 0], (0, pad)).reshape(ROWS, LANES)
    res = jnp.pad(res_scores, (0, pad)).reshape(ROWS, LANES)
    reg = jnp.pad(box_regression, ((0, pad), (0, 0))).T.reshape(7, ROWS, LANES)
    anc = jnp.pad(anchors, ((0, pad), (0, 0))).T.reshape(7, ROWS, LANES)

    table, destt, destr = _prep(resid, logit, res, reg, anc)
    table2d = table.reshape(NCH, NPAD).T
    rows = _scatter_rows(table2d,
                         destt.reshape(NW, EPW // LANES, LANES),
                         destr.reshape(NW, EPW // LANES, LANES))

    tcol = rows[:KPAD]
    rcol = rows[R_OFF:R_OFF + KPAD]
    out = _nms(tcol, tcol.T, rcol, rcol.T)
    return out[:, :POST, :]
